# baseline (speedup 1.0000x reference)
import jax
import jax.numpy as jnp
from jax import lax
from jax.experimental import pallas as pl
from jax.experimental.pallas import tpu as pltpu
from jax.experimental.pallas import tpu_sc as plsc

D = 32
DP = 128
NC, NS = 2, 16
NW = NC * NS
SEQ = 50
BATCH = 16384
TOK = 256
NBLK = SEQ * (BATCH // TOK)
PER_W = NBLK // NW
NBUF = 2
TB_WORDS = TOK * D
OUT_WORDS = SEQ * 4 * (BATCH // 128) * 8 * 128


def _emb_body(tok_hbm, table_hbm, out_hbm, idx_v, buf_v, tbuf_v,
              isems, gsems, ssems):
    wid = lax.axis_index("s") * NC + lax.axis_index("c")
    lane = lax.iota(jnp.int32, 16)
    p0 = ((lane >> 3) << 11) + ((lane & 7) << 7)
    p1 = p0 + 4096

    def sb(m):
        return m >> 6, m & 63

    def load_ids(m, b):
        s, b4 = sb(m)
        return pltpu.async_copy(
            tok_hbm.at[s, pl.ds(b4 * TOK, TOK)], idx_v.at[b], isems[b])

    def gather(b):
        return pltpu.async_copy(
            table_hbm.at[idx_v.at[b]], buf_v.at[b], gsems[b])

    def wait_ids(b):
        pltpu.make_async_copy(
            tok_hbm.at[0, pl.ds(0, TOK)], idx_v.at[b], isems[b]).wait()

    def wait_gather(b):
        pltpu.make_async_copy(
            table_hbm.at[idx_v.at[b]], buf_v.at[b], gsems[b]).wait()

    def wait_stores(b):
        for k in range(4):
            pltpu.make_async_copy(
                tbuf_v.at[b, pl.ds(k * 2048, 2048)],
                out_hbm.at[pl.ds(0, 2048)], ssems[b]).wait()

    def transpose_and_store(m, b):
        s, b4 = sb(m)

        for bq in range(2):
            q0 = p0 + (bq << 10)
            q1 = p1 + (bq << 10)
            cb = bq << 7

            @plsc.parallel_loop(0, 128, unroll=8)
            def _(cc, q0=q0, q1=q1, cb=cb):
                c = cb + cc
                v0 = buf_v[b, c, pl.ds(0, 16)]
                plsc.store_scatter(tbuf_v.at[b], [q0 + cc], v0)
                v1 = buf_v[b, c, pl.ds(16, 16)]
                plsc.store_scatter(tbuf_v.at[b], [q1 + cc], v1)

        out0 = (s * 4 * 128 + b4 * 2) * 1024
        for k in range(4):
            pltpu.async_copy(
                tbuf_v.at[b, pl.ds(k * 2048, 2048)],
                out_hbm.at[pl.ds(out0 + k * 131072, 2048)], ssems[b])

    m0 = wid * PER_W
    load_ids(m0, 0)
    wait_ids(0)
    gather(0)
    load_ids(m0 + 1, 1)

    def group(j, _):
        for b in (0, 1):
            i = j * 2 + b
            nb = 1 - b
            wait_gather(b)

            @pl.when(i < PER_W - 1)
            def _():
                wait_ids(nb)
                gather(nb)

            @pl.when(i >= 2)
            def _():
                wait_stores(b)

            transpose_and_store(m0 + i, b)

            @pl.when(i < PER_W - 2)
            def _():
                load_ids(m0 + i + 2, b)
        return 0

    lax.fori_loop(0, PER_W // 2, group, 0)
    wait_stores(0)
    wait_stores(1)


def kernel(token_ids, weights):
    tok_t = token_ids.T
    w128 = jnp.pad(weights, ((0, 0), (0, DP - D)))

    mesh = plsc.VectorSubcoreMesh(
        core_axis_name="c", subcore_axis_name="s", num_cores=NC, num_subcores=NS
    )
    grab = pl.kernel(
        _emb_body,
        out_type=jax.ShapeDtypeStruct((OUT_WORDS,), jnp.float32),
        mesh=mesh,
        scratch_types=[
            pltpu.VMEM((NBUF, TOK), jnp.int32),
            pltpu.VMEM((NBUF, TOK, DP), jnp.float32),
            pltpu.VMEM((NBUF, 4 * 2048), jnp.float32),
            [pltpu.SemaphoreType.DMA] * NBUF,
            [pltpu.SemaphoreType.DMA] * NBUF,
            [pltpu.SemaphoreType.DMA] * NBUF,
        ],
        compiler_params=pltpu.CompilerParams(
            use_tc_tiling_on_sc=False, needs_layout_passes=False),
    )
    out1 = grab(tok_t, w128)
    out5 = out1.reshape(SEQ, 4, BATCH // 128, 8, 128)
    return out5.transpose(2, 4, 0, 1, 3).reshape(BATCH, SEQ, D)


# speedup vs baseline: 1.0277x; 1.0277x over previous
import jax
import jax.numpy as jnp
from jax import lax
from jax.experimental import pallas as pl
from jax.experimental.pallas import tpu as pltpu
from jax.experimental.pallas import tpu_sc as plsc

D = 32
NC, NS = 2, 16
NW = NC * NS
SEQ = 50
BATCH = 16384
TOK = 512
NBLK = SEQ * (BATCH // TOK)
PER_W = NBLK // NW
NBUF = 2
BLK_WORDS = TOK * D
OUT_WORDS = SEQ * 4 * (BATCH // 128) * 8 * 128


def _emb_body(tok_hbm, table_hbm, out_hbm, idx_v, buf_v, tbuf_v,
              isems, gsems, ssems):
    wid = lax.axis_index("s") * NC + lax.axis_index("c")
    lane = lax.iota(jnp.int32, 16)
    p0 = ((lane >> 3) << 12) + ((lane & 7) << 7)
    p1 = p0 + 8192

    def sb(m):
        return m >> 5, m & 31

    def load_ids(m, b):
        s, b4 = sb(m)
        return pltpu.async_copy(
            tok_hbm.at[s, pl.ds(b4 * TOK, TOK)], idx_v.at[b], isems[b])

    def gather(b):
        return pltpu.async_copy(
            table_hbm.at[idx_v.at[b]], buf_v.at[b], gsems[b])

    def wait_ids(b):
        pltpu.make_async_copy(
            tok_hbm.at[0, pl.ds(0, TOK)], idx_v.at[b], isems[b]).wait()

    def wait_gather(b):
        pltpu.make_async_copy(
            table_hbm.at[idx_v.at[b]], buf_v.at[b], gsems[b]).wait()

    def wait_stores(b):
        for k in range(4):
            pltpu.make_async_copy(
                tbuf_v.at[b, pl.ds(k * 4096, 4096)],
                out_hbm.at[pl.ds(0, 4096)], ssems[b]).wait()

    def transpose_and_store(m, b):
        s, b4 = sb(m)

        for bq in range(4):
            q0 = p0 + (bq << 10)
            q1 = p1 + (bq << 10)
            cb = bq << 7

            @plsc.parallel_loop(0, 128, unroll=16)
            def _(cc, q0=q0, q1=q1, cb=cb):
                c = cb + cc
                v0 = buf_v[b, c, pl.ds(0, 16)]
                plsc.store_scatter(tbuf_v.at[b], [q0 + cc], v0)
                v1 = buf_v[b, c, pl.ds(16, 16)]
                plsc.store_scatter(tbuf_v.at[b], [q1 + cc], v1)
        out0 = (s * 4 * 128 + b4 * 4) * 1024
        for k in range(4):
            pltpu.async_copy(
                tbuf_v.at[b, pl.ds(k * 4096, 4096)],
                out_hbm.at[pl.ds(out0 + k * 131072, 4096)], ssems[b])

    m0 = wid * PER_W
    load_ids(m0, 0)
    wait_ids(0)
    gather(0)
    load_ids(m0 + 1, 1)

    def group(j, _):
        for b in (0, 1):
            i = j * 2 + b
            nb = 1 - b
            wait_gather(b)

            @pl.when(i < PER_W - 1)
            def _():
                wait_ids(nb)
                gather(nb)

            @pl.when(i >= 2)
            def _():
                wait_stores(b)

            transpose_and_store(m0 + i, b)

            @pl.when(i < PER_W - 2)
            def _():
                load_ids(m0 + i + 2, b)
        return 0

    lax.fori_loop(0, PER_W // 2, group, 0)
    wait_stores(0)
    wait_stores(1)


def kernel(token_ids, weights):
    tok_t = token_ids.T

    mesh = plsc.VectorSubcoreMesh(
        core_axis_name="c", subcore_axis_name="s", num_cores=NC, num_subcores=NS
    )
    grab = pl.kernel(
        _emb_body,
        out_type=jax.ShapeDtypeStruct((OUT_WORDS,), jnp.float32),
        mesh=mesh,
        scratch_types=[
            pltpu.VMEM((NBUF, TOK), jnp.int32),
            pltpu.VMEM((NBUF, TOK, D), jnp.float32),
            pltpu.VMEM((NBUF, 4 * 4096), jnp.float32),
            [pltpu.SemaphoreType.DMA] * NBUF,
            [pltpu.SemaphoreType.DMA] * NBUF,
            [pltpu.SemaphoreType.DMA] * NBUF,
        ],
        compiler_params=pltpu.CompilerParams(
            use_tc_tiling_on_sc=False, needs_layout_passes=False),
    )
    out1 = grab(tok_t, weights)
    out5 = out1.reshape(SEQ, 4, BATCH // 128, 8, 128)
    return out5.transpose(2, 4, 0, 1, 3).reshape(BATCH, SEQ, D)
